# baseline (speedup 1.0000x reference)
import jax
import jax.numpy as jnp
from jax.experimental import pallas as pl
from jax.experimental.pallas import tpu as pltpu


def _fused_block(x_ref, w_ref, b_ref, o_ref, m_ref, b2_ref):
    @pl.when(pl.program_id(0) == 0)
    def _():
        a = w_ref.shape[0]
        row = jax.lax.broadcasted_iota(jnp.int32, (a, a), 0)
        col = jax.lax.broadcasted_iota(jnp.int32, (a, a), 1)
        eye = jnp.where(row == col, jnp.float32(1.0), jnp.float32(0.0))
        m_ref[...] = w_ref[...] * 0.1 + eye
        b2_ref[...] = b_ref[...] * 0.1

    y = jax.lax.dot_general(
        x_ref[...], m_ref[...], (((1,), (1,)), ((), ())),
        preferred_element_type=jnp.float32,
    )
    o_ref[...] = y + b2_ref[...]


def kernel(x, W, b):
    n, a = x.shape
    tm = 2000
    grid = (n // tm,)
    return pl.pallas_call(
        _fused_block,
        grid=grid,
        in_specs=[
            pl.BlockSpec((tm, a), lambda i: (i, 0)),
            pl.BlockSpec((a, a), lambda i: (0, 0)),
            pl.BlockSpec((1, a), lambda i: (0, 0)),
        ],
        out_specs=pl.BlockSpec((tm, a), lambda i: (i, 0)),
        out_shape=jax.ShapeDtypeStruct((n, a), jnp.float32),
        scratch_shapes=[
            pltpu.VMEM((a, a), jnp.float32),
            pltpu.VMEM((1, a), jnp.float32),
        ],
        compiler_params=pltpu.CompilerParams(
            dimension_semantics=("arbitrary",),
        ),
    )(x, W, b.reshape(1, a))


# speedup vs baseline: 1.1605x; 1.1605x over previous
import jax
import jax.numpy as jnp
from jax.experimental import pallas as pl
from jax.experimental.pallas import tpu as pltpu


def _fused_block(x_ref, w_ref, b_ref, o_ref, m_ref, b2_ref):
    @pl.when(pl.program_id(0) == 0)
    def _():
        a = w_ref.shape[0]
        row = jax.lax.broadcasted_iota(jnp.int32, (a, a), 0)
        col = jax.lax.broadcasted_iota(jnp.int32, (a, a), 1)
        eye = jnp.where(row == col, jnp.float32(1.0), jnp.float32(0.0))
        m_ref[...] = w_ref[...] * 0.1 + eye
        b2_ref[...] = b_ref[...] * 0.1

    y = jax.lax.dot_general(
        x_ref[...], m_ref[...], (((1,), (1,)), ((), ())),
        preferred_element_type=jnp.float32,
    )
    o_ref[...] = y + b2_ref[...]


def kernel(x, W, b):
    n, a = x.shape
    tm = 5000
    grid = (n // tm,)
    return pl.pallas_call(
        _fused_block,
        grid=grid,
        in_specs=[
            pl.BlockSpec((tm, a), lambda i: (i, 0)),
            pl.BlockSpec((a, a), lambda i: (0, 0)),
            pl.BlockSpec((1, a), lambda i: (0, 0)),
        ],
        out_specs=pl.BlockSpec((tm, a), lambda i: (i, 0)),
        out_shape=jax.ShapeDtypeStruct((n, a), jnp.float32),
        scratch_shapes=[
            pltpu.VMEM((a, a), jnp.float32),
            pltpu.VMEM((1, a), jnp.float32),
        ],
        compiler_params=pltpu.CompilerParams(
            dimension_semantics=("arbitrary",),
        ),
    )(x, W, b.reshape(1, a))
